# Initial kernel scaffold; baseline (speedup 1.0000x reference)
import functools

import jax
import jax.numpy as jnp
from jax import lax
from jax.experimental import pallas as pl
from jax.experimental.pallas import tpu as pltpu
from jax.experimental.pallas import tpu_sc as plsc

N = 10000
D = 256
H = 128
E = 160000
NC = 2
NS = 16
C = 80
NR = N // NS
EP = E // NS

R = 1000
NB = N // R


def _sc_aggregate_body(y_st, src_h, dst_h, ew_h, rst_st,
                       agg_sh, src_v, dst_v, ew_v, rows_v, sem):
    c = lax.axis_index("c")
    s = lax.axis_index("s")
    row0 = c * N + s * NR
    pltpu.sync_copy(y_st.at[pl.ds(row0, NR)], agg_sh.at[pl.ds(s * NR, NR)])
    plsc.subcore_barrier()

    base0 = s * EP
    tbl_off = c * N

    @pl.loop(0, EP // C)
    def _chunk(k):
        base = base0 + k * C
        pltpu.sync_copy(src_h.at[pl.ds(base, C)], src_v)
        pltpu.sync_copy(dst_h.at[pl.ds(base, C)], dst_v)
        pltpu.sync_copy(ew_h.at[pl.ds(base, C)], ew_v)

        @pl.loop(0, C // 16)
        def _off(i):
            sl = pl.ds(i * 16, 16)
            src_v[sl] = src_v[sl] + tbl_off

        pltpu.async_copy(y_st.at[src_v], rows_v, sem).wait()

        @pl.loop(0, C)
        def _scale(e):
            w = ew_v[e]
            for j in range(H // 16):
                sl = pl.ds(j * 16, 16)
                rows_v[e, sl] = rows_v[e, sl] * w

        pltpu.sync_copy(rows_v, agg_sh.at[dst_v], add=True)

    plsc.subcore_barrier()
    pltpu.sync_copy(agg_sh.at[pl.ds(s * NR, NR)], rst_st.at[pl.ds(row0, NR)])


@functools.cache
def _build_sc_aggregate():
    mesh = plsc.VectorSubcoreMesh(core_axis_name="c", subcore_axis_name="s",
                                  num_cores=NC, num_subcores=NS)
    return pl.kernel(
        _sc_aggregate_body,
        out_type=jax.ShapeDtypeStruct((NC * N, H), jnp.float32),
        mesh=mesh,
        scratch_types=[
            pltpu.VMEM_SHARED((N, H), jnp.float32),
            pltpu.VMEM((C,), jnp.int32),
            pltpu.VMEM((C,), jnp.int32),
            pltpu.VMEM((C,), jnp.float32),
            pltpu.VMEM((C, H), jnp.float32),
            pltpu.SemaphoreType.DMA,
        ],
    )


def _tc_linear_body(lo, hi, wlo, whi, b, out):
    acc = jnp.dot(lo[...], wlo[...], preferred_element_type=jnp.float32)
    acc += jnp.dot(hi[...], whi[...], preferred_element_type=jnp.float32)
    out[...] = jnp.maximum(acc + b[...], 0.0)


def _tc_mean_body(lo, hi, wlo, whi, b, out):
    i = pl.program_id(1)
    acc = jnp.dot(lo[...], wlo[...], preferred_element_type=jnp.float32)
    acc += jnp.dot(hi[...], whi[...], preferred_element_type=jnp.float32)
    x2 = jnp.maximum(acc + b[...], 0.0)
    ssum = jnp.sum(x2, axis=0, keepdims=True)

    @pl.when(i == 0)
    def _():
        out[...] = jnp.zeros_like(out)

    out[...] += ssum

    @pl.when(i == NB - 1)
    def _():
        out[...] = out[...] * (1.0 / N)


_IN_SPECS = [
    pl.BlockSpec((R, H), lambda j, i: (i, 0)),
    pl.BlockSpec((R, H), lambda j, i: (NB + i, 0)),
    pl.BlockSpec((H, H), lambda j, i: (0, j)),
    pl.BlockSpec((H, H), lambda j, i: (1, j)),
    pl.BlockSpec((1, H), lambda j, i: (0, j)),
]


@jax.jit
def _tc_linear(rst_st, wt, b2):
    return pl.pallas_call(
        _tc_linear_body,
        grid=(2, NB),
        in_specs=_IN_SPECS,
        out_specs=pl.BlockSpec((R, H), lambda j, i: (j * NB + i, 0)),
        out_shape=jax.ShapeDtypeStruct((NC * N, H), jnp.float32),
    )(rst_st, rst_st, wt, wt, b2)


@jax.jit
def _tc_mean(rst_st, wt, b2):
    return pl.pallas_call(
        _tc_mean_body,
        grid=(2, NB),
        in_specs=_IN_SPECS,
        out_specs=pl.BlockSpec((1, H), lambda j, i: (0, j)),
        out_shape=jax.ShapeDtypeStruct((1, D), jnp.float32),
    )(rst_st, rst_st, wt, wt, b2)


def kernel(h, edge_index, edge_weight, W, b):
    src = edge_index[0].astype(jnp.int32)
    dst = edge_index[1].astype(jnp.int32)
    ew = edge_weight.astype(jnp.float32)
    h_st = jnp.concatenate([h[:, :H], h[:, H:]], axis=0)
    wt = W.T
    b2 = b.reshape(1, D)

    sc_aggregate = _build_sc_aggregate()
    rst1 = sc_aggregate(h_st, src, dst, ew)
    x_st = _tc_linear(rst1, wt, b2)
    rst2 = sc_aggregate(x_st, src, dst, ew)
    return _tc_mean(rst2, wt, b2)


# speedup vs baseline: 2.6605x; 2.6605x over previous
import functools

import jax
import jax.numpy as jnp
from jax import lax
from jax.experimental import pallas as pl
from jax.experimental.pallas import tpu as pltpu
from jax.experimental.pallas import tpu_sc as plsc

N = 10000
D = 256
H = 128
E = 160000
NC = 2
NS = 16
C = 80
NR = 624
REM = N - NS * NR
EP = E // NS

R = 1000
NB = N // R


def _sc_aggregate_body(y_st, src_h, dst_h, ew_h, rst_st,
                       agg_sh, src_v, dst_v, ew_v, rows_v, sem):
    c = lax.axis_index("c")
    s = lax.axis_index("s")
    row0 = c * N + s * NR
    pltpu.sync_copy(y_st.at[pl.ds(row0, NR)], agg_sh.at[pl.ds(s * NR, NR)])

    @pl.when(s == NS - 1)
    def _seed_rem():
        pltpu.sync_copy(y_st.at[pl.ds(c * N + NS * NR, REM)],
                        agg_sh.at[pl.ds(NS * NR, REM)])

    plsc.subcore_barrier()

    base0 = s * EP
    tbl_off = c * N

    @pl.loop(0, EP // C)
    def _chunk(k):
        base = base0 + k * C
        pltpu.sync_copy(src_h.at[pl.ds(base, C)], src_v)
        pltpu.sync_copy(dst_h.at[pl.ds(base, C)], dst_v)
        pltpu.sync_copy(ew_h.at[pl.ds(base, C)], ew_v)

        @pl.loop(0, C // 16)
        def _off(i):
            sl = pl.ds(i * 16, 16)
            src_v[sl] = src_v[sl] + tbl_off

        pltpu.async_copy(y_st.at[src_v], rows_v, sem).wait()

        @pl.loop(0, C // 16)
        def _scale(g):
            w16 = ew_v[pl.ds(g * 16, 16)]
            for e in range(16):
                w = w16[e]
                for j in range(H // 16):
                    sl = pl.ds(j * 16, 16)
                    rows_v[g * 16 + e, sl] = rows_v[g * 16 + e, sl] * w

        pltpu.sync_copy(rows_v, agg_sh.at[dst_v], add=True)

    plsc.subcore_barrier()
    pltpu.sync_copy(agg_sh.at[pl.ds(s * NR, NR)], rst_st.at[pl.ds(row0, NR)])

    @pl.when(s == NS - 1)
    def _write_rem():
        pltpu.sync_copy(agg_sh.at[pl.ds(NS * NR, REM)],
                        rst_st.at[pl.ds(c * N + NS * NR, REM)])


@functools.cache
def _build_sc_aggregate():
    mesh = plsc.VectorSubcoreMesh(core_axis_name="c", subcore_axis_name="s",
                                  num_cores=NC, num_subcores=NS)
    return pl.kernel(
        _sc_aggregate_body,
        out_type=jax.ShapeDtypeStruct((NC * N, H), jnp.float32),
        mesh=mesh,
        scratch_types=[
            pltpu.VMEM_SHARED((N, H), jnp.float32),
            pltpu.VMEM((C,), jnp.int32),
            pltpu.VMEM((C,), jnp.int32),
            pltpu.VMEM((C,), jnp.float32),
            pltpu.VMEM((C, H), jnp.float32),
            pltpu.SemaphoreType.DMA,
        ],
    )


def _tc_linear_body(lo, hi, wlo, whi, b, out):
    acc = jnp.dot(lo[...], wlo[...], preferred_element_type=jnp.float32)
    acc += jnp.dot(hi[...], whi[...], preferred_element_type=jnp.float32)
    out[...] = jnp.maximum(acc + b[...], 0.0)


def _tc_mean_body(lo, hi, wlo, whi, b, out):
    i = pl.program_id(1)
    acc = jnp.dot(lo[...], wlo[...], preferred_element_type=jnp.float32)
    acc += jnp.dot(hi[...], whi[...], preferred_element_type=jnp.float32)
    x2 = jnp.maximum(acc + b[...], 0.0)
    ssum = jnp.sum(x2, axis=0, keepdims=True)

    @pl.when(i == 0)
    def _():
        out[...] = jnp.zeros_like(out)

    out[...] += ssum

    @pl.when(i == NB - 1)
    def _():
        out[...] = out[...] * (1.0 / N)


_IN_SPECS = [
    pl.BlockSpec((R, H), lambda j, i: (i, 0)),
    pl.BlockSpec((R, H), lambda j, i: (NB + i, 0)),
    pl.BlockSpec((H, H), lambda j, i: (0, j)),
    pl.BlockSpec((H, H), lambda j, i: (1, j)),
    pl.BlockSpec((1, H), lambda j, i: (0, j)),
]


@jax.jit
def _tc_linear(rst_st, wt, b2):
    return pl.pallas_call(
        _tc_linear_body,
        grid=(2, NB),
        in_specs=_IN_SPECS,
        out_specs=pl.BlockSpec((R, H), lambda j, i: (j * NB + i, 0)),
        out_shape=jax.ShapeDtypeStruct((NC * N, H), jnp.float32),
    )(rst_st, rst_st, wt, wt, b2)


@jax.jit
def _tc_mean(rst_st, wt, b2):
    return pl.pallas_call(
        _tc_mean_body,
        grid=(2, NB),
        in_specs=_IN_SPECS,
        out_specs=pl.BlockSpec((1, H), lambda j, i: (0, j)),
        out_shape=jax.ShapeDtypeStruct((1, D), jnp.float32),
    )(rst_st, rst_st, wt, wt, b2)


def kernel(h, edge_index, edge_weight, W, b):
    src = edge_index[0].astype(jnp.int32)
    dst = edge_index[1].astype(jnp.int32)
    ew = edge_weight.astype(jnp.float32)
    h_st = jnp.concatenate([h[:, :H], h[:, H:]], axis=0)
    wt = W.T
    b2 = b.reshape(1, D)

    sc_aggregate = _build_sc_aggregate()
    rst1 = sc_aggregate(h_st, src, dst, ew)
    x_st = _tc_linear(rst1, wt, b2)
    rst2 = sc_aggregate(x_st, src, dst, ew)
    return _tc_mean(rst2, wt, b2)


# speedup vs baseline: 3.7083x; 1.3938x over previous
import functools

import jax
import jax.numpy as jnp
from jax import lax
from jax.experimental import pallas as pl
from jax.experimental.pallas import tpu as pltpu
from jax.experimental.pallas import tpu_sc as plsc

N = 10000
D = 256
H = 128
E = 160000
NC = 2
NS = 16
C = 64
NR = 624
REM = N - NS * NR
EPP = 10240
NCH = EPP // C
NBUF = 5

R = 1000
NB = N // R


def _sc_aggregate_body(y_st, eidx, eww, rst_st,
                       agg_sh, idxb, ewb, rows, isem, wsem, gsem, ssem):
    c = lax.axis_index("c")
    s = lax.axis_index("s")
    row0 = c * N + s * NR
    tbl_off = c * N

    def start_idx(k, b):
        pltpu.async_copy(eidx.at[s * NCH + k], idxb.at[b], isem.at[b])
        pltpu.async_copy(eww.at[s * NCH + k], ewb.at[b], wsem.at[b])

    def wait_idx(k, b):
        pltpu.make_async_copy(eidx.at[s * NCH + k], idxb.at[b],
                              isem.at[b]).wait()
        pltpu.make_async_copy(eww.at[s * NCH + k], ewb.at[b],
                              wsem.at[b]).wait()
        for g in range(C // 16):
            sl = pl.ds(g * 16, 16)
            idxb[b, 0, sl] = idxb[b, 0, sl] + tbl_off

    def start_gather(k, b):
        pltpu.async_copy(y_st.at[idxb.at[b, 0]], rows.at[b], gsem.at[b])

    def wait_gather(k, b):
        pltpu.make_async_copy(y_st.at[idxb.at[b, 0]], rows.at[b],
                              gsem.at[b]).wait()

    def do_scale(k, b):
        @pl.loop(0, C // 16)
        def _scale(g):
            w16 = ewb[b, pl.ds(g * 16, 16)]
            for e in range(16):
                w = w16[e]
                for j in range(H // 16):
                    sl = pl.ds(j * 16, 16)
                    rows[b, g * 16 + e, sl] = rows[b, g * 16 + e, sl] * w

    def start_scatter(k, b):
        pltpu.async_copy(rows.at[b], agg_sh.at[idxb.at[b, 1]], ssem.at[b],
                         add=True)

    def wait_scatter(k, b):
        pltpu.make_async_copy(rows.at[b], agg_sh.at[idxb.at[b, 1]],
                              ssem.at[b]).wait()

    for k in range(3):
        start_idx(k, k)
    for k in range(2):
        wait_idx(k, k)
        start_gather(k, k)

    pltpu.sync_copy(y_st.at[pl.ds(row0, NR)], agg_sh.at[pl.ds(s * NR, NR)])

    @pl.when(s == NS - 1)
    def _seed_rem():
        pltpu.sync_copy(y_st.at[pl.ds(c * N + NS * NR, REM)],
                        agg_sh.at[pl.ds(NS * NR, REM)])

    plsc.subcore_barrier()

    for k in range(NBUF):
        wait_gather(k, k)
        do_scale(k, k)
        start_scatter(k, k)
        if k >= 2:
            wait_scatter(k - 2, (k + 3) % NBUF)
        start_idx(k + 3, (k + 3) % NBUF)
        wait_idx(k + 2, (k + 2) % NBUF)
        start_gather(k + 2, (k + 2) % NBUF)

    @pl.loop(1, NCH // NBUF)
    def _grp(kk):
        for par in range(NBUF):
            k = kk * NBUF + par
            wait_gather(k, par)
            do_scale(k, par)
            start_scatter(k, par)
            b3 = (par + 3) % NBUF
            b2 = (par + 2) % NBUF

            @pl.when(k + 3 < NCH)
            def _refill_idx():
                wait_scatter(k - 2, b3)
                start_idx(k + 3, b3)

            @pl.when(k + 2 < NCH)
            def _refill_gather():
                wait_idx(k + 2, b2)
                start_gather(k + 2, b2)

    for b in range(NBUF):
        wait_scatter(NCH - NBUF + b, b)

    plsc.subcore_barrier()
    pltpu.sync_copy(agg_sh.at[pl.ds(s * NR, NR)], rst_st.at[pl.ds(row0, NR)])

    @pl.when(s == NS - 1)
    def _write_rem():
        pltpu.sync_copy(agg_sh.at[pl.ds(NS * NR, REM)],
                        rst_st.at[pl.ds(c * N + NS * NR, REM)])


@functools.cache
def _build_sc_aggregate():
    mesh = plsc.VectorSubcoreMesh(core_axis_name="c", subcore_axis_name="s",
                                  num_cores=NC, num_subcores=NS)
    return pl.kernel(
        _sc_aggregate_body,
        out_type=jax.ShapeDtypeStruct((NC * N, H), jnp.float32),
        mesh=mesh,
        scratch_types=[
            pltpu.VMEM_SHARED((N, H), jnp.float32),
            pltpu.VMEM((NBUF, 2, C), jnp.int32),
            pltpu.VMEM((NBUF, C), jnp.float32),
            pltpu.VMEM((NBUF, C, H), jnp.float32),
            pltpu.SemaphoreType.DMA((NBUF,)),
            pltpu.SemaphoreType.DMA((NBUF,)),
            pltpu.SemaphoreType.DMA((NBUF,)),
            pltpu.SemaphoreType.DMA((NBUF,)),
        ],
    )


def _tc_linear_body(lo, hi, wlo, whi, b, out):
    acc = jnp.dot(lo[...], wlo[...], preferred_element_type=jnp.float32)
    acc += jnp.dot(hi[...], whi[...], preferred_element_type=jnp.float32)
    out[...] = jnp.maximum(acc + b[...], 0.0)


def _tc_mean_body(lo, hi, wlo, whi, b, out):
    i = pl.program_id(1)
    acc = jnp.dot(lo[...], wlo[...], preferred_element_type=jnp.float32)
    acc += jnp.dot(hi[...], whi[...], preferred_element_type=jnp.float32)
    x2 = jnp.maximum(acc + b[...], 0.0)
    ssum = jnp.sum(x2, axis=0, keepdims=True)

    @pl.when(i == 0)
    def _():
        out[...] = jnp.zeros_like(out)

    out[...] += ssum

    @pl.when(i == NB - 1)
    def _():
        out[...] = out[...] * (1.0 / N)


_IN_SPECS = [
    pl.BlockSpec((R, H), lambda j, i: (i, 0)),
    pl.BlockSpec((R, H), lambda j, i: (NB + i, 0)),
    pl.BlockSpec((H, H), lambda j, i: (0, j)),
    pl.BlockSpec((H, H), lambda j, i: (1, j)),
    pl.BlockSpec((1, H), lambda j, i: (0, j)),
]


@jax.jit
def _tc_linear(rst_st, wt, b2):
    return pl.pallas_call(
        _tc_linear_body,
        grid=(2, NB),
        in_specs=_IN_SPECS,
        out_specs=pl.BlockSpec((R, H), lambda j, i: (j * NB + i, 0)),
        out_shape=jax.ShapeDtypeStruct((NC * N, H), jnp.float32),
    )(rst_st, rst_st, wt, wt, b2)


@jax.jit
def _tc_mean(rst_st, wt, b2):
    return pl.pallas_call(
        _tc_mean_body,
        grid=(2, NB),
        in_specs=_IN_SPECS,
        out_specs=pl.BlockSpec((1, H), lambda j, i: (0, j)),
        out_shape=jax.ShapeDtypeStruct((1, D), jnp.float32),
    )(rst_st, rst_st, wt, wt, b2)


def kernel(h, edge_index, edge_weight, W, b):
    pad = NS * EPP - E
    src = jnp.pad(edge_index[0].astype(jnp.int32).reshape(NS, E // NS),
                  ((0, 0), (0, pad // NS)))
    dst = jnp.pad(edge_index[1].astype(jnp.int32).reshape(NS, E // NS),
                  ((0, 0), (0, pad // NS)))
    eww = jnp.pad(edge_weight.astype(jnp.float32).reshape(NS, E // NS),
                  ((0, 0), (0, pad // NS))).reshape(NS * NCH, C)
    eidx = jnp.stack(
        [src.reshape(NS, NCH, C), dst.reshape(NS, NCH, C)],
        axis=2).reshape(NS * NCH, 2, C)
    h_st = jnp.concatenate([h[:, :H], h[:, H:]], axis=0)
    wt = W.T
    b2 = b.reshape(1, D)

    sc_aggregate = _build_sc_aggregate()
    rst1 = sc_aggregate(h_st, eidx, eww)
    x_st = _tc_linear(rst1, wt, b2)
    rst2 = sc_aggregate(x_st, eidx, eww)
    return _tc_mean(rst2, wt, b2)
